# baseline (speedup 1.0000x reference)
import functools

import jax
import jax.numpy as jnp
from jax import lax
from jax.experimental import pallas as pl
from jax.experimental.pallas import tpu as pltpu
from jax.experimental.pallas import tpu_sc as plsc

N = 10000
E = 160000
D = 256
H = 256
C = 10
K = 10
ALPHA = 0.1

NW = 16
E_PER = E // NW
N_PAD = 10240
S_PER = N_PAD // NW
L = 16


def _rsqrt16(x):
    i = plsc.bitcast(x, jnp.int32)
    i = jnp.int32(0x5F3759DF) - jnp.right_shift(i, 1)
    y = plsc.bitcast(i, jnp.float32)
    for _ in range(3):
        y = y * (1.5 - 0.5 * x * y * y)
    return y


NR = 80
RPW = NR // NW
E_MAIN = 128 * 78
EC_MAIN = E_MAIN // L
EW = E_MAIN + 128


def _sc_propagate_body(ei_hbm, w_hbm,
                       ei_v, we_v, v_v, acc_v, nrm_v,
                       idx80_v, zrows_v, wsl_v, buf0_sh, buf1_sh, buf2_sh,
                       dma_sem, dma_sem2):
    wid = lax.axis_index("s")
    rbase = wid * RPW
    SC_ = S_PER // L
    bufs = [buf0_sh, buf1_sh, buf2_sh]

    cp_ei = pltpu.async_copy(
        ei_hbm.at[:, pl.ds(pl.multiple_of(wid * E_MAIN, 128), E_MAIN)],
        ei_v.at[:, pl.ds(0, E_MAIN)], dma_sem)

    @pl.when(wid < 2)
    def _():
        pltpu.sync_copy(
            ei_hbm.at[:, pl.ds(pl.multiple_of(16 * E_MAIN + wid * 128, 128),
                               128)],
            ei_v.at[:, pl.ds(E_MAIN, 128)])

    def edges_loop(body, unroll=4):
        plsc.parallel_loop(0, EC_MAIN, unroll=unroll)(body)

        @pl.when(wid < 2)
        def _():
            plsc.parallel_loop(EC_MAIN, EC_MAIN + 8, unroll=8)(body)

    zeros16 = jnp.zeros((L,), jnp.float32)
    ones16 = jnp.ones((L,), jnp.float32)

    def zero_acc():
        @plsc.parallel_loop(0, NR * 8, unroll=8)
        def _(i):
            acc_v[i // 8, pl.ds((i % 8) * L, L)] = zeros16

    for j in range(RPW):
        idx80_v[pl.ds(j * L, L)] = lax.iota(jnp.int32, L) + j * L

    @plsc.parallel_loop(0, RPW * 8, unroll=8)
    def _(i):
        zrows_v[i // 8, pl.ds((i % 8) * L, L)] = zeros16

    zero_acc()
    pltpu.sync_copy(zrows_v, buf0_sh.at[pl.ds(rbase, RPW)])
    pltpu.sync_copy(zrows_v, buf1_sh.at[pl.ds(rbase, RPW)])

    cp_ei.wait()

    def _deg_body(i):
        d = ei_v[1, pl.ds(i * L, L)]
        drow = jnp.right_shift(d, 7)
        dcol = jnp.bitwise_and(d, 127)
        plsc.addupdate_scatter(acc_v, [drow, dcol], ones16)

    edges_loop(_deg_body)

    plsc.subcore_barrier()
    pltpu.sync_copy(acc_v, buf0_sh.at[idx80_v], add=True)
    plsc.subcore_barrier()
    cp_deg = pltpu.async_copy(buf0_sh, v_v, dma_sem)
    zero_acc()
    cp_deg.wait()

    @plsc.parallel_loop(0, NR * 8, unroll=4)
    def _(i):
        r = i // 8
        c = (i % 8) * L
        nrm_v[r, pl.ds(c, L)] = _rsqrt16(jnp.maximum(v_v[r, pl.ds(c, L)], 1.0))

    def _we_body(i):
        s = ei_v[0, pl.ds(i * L, L)]
        d = ei_v[1, pl.ds(i * L, L)]
        srow = jnp.right_shift(s, 7)
        scol = jnp.bitwise_and(s, 127)
        drow = jnp.right_shift(d, 7)
        dcol = jnp.bitwise_and(d, 127)
        we = (plsc.load_gather(nrm_v, [srow, scol]) *
              plsc.load_gather(nrm_v, [drow, dcol]))
        we_v[pl.ds(i * L, L)] = we
        plsc.addupdate_scatter(acc_v, [srow, scol], we * (1.0 / N))

    edges_loop(_we_body)

    cp_add1 = pltpu.async_copy(acc_v, buf1_sh.at[idx80_v], dma_sem, add=True)
    pltpu.sync_copy(zrows_v, buf2_sh.at[pl.ds(rbase, RPW)])
    cp_add1.wait()
    plsc.subcore_barrier()
    cp_rd1 = pltpu.async_copy(buf1_sh, v_v, dma_sem2)
    zero_acc()
    cp_rd1.wait()

    cdamp = 1.0 - ALPHA
    coef = ALPHA * cdamp if K > 1 else cdamp

    @plsc.parallel_loop(0, SC_, unroll=5)
    def _(j):
        r = rbase + j // 8
        c = (j % 8) * L
        wsl_v[pl.ds(j * L, L)] = ALPHA * (1.0 / N) + coef * v_v[r, pl.ds(c, L)]

    for it in range(2, K + 1):
        out_sh = bufs[it % 3]
        nxt_sh = bufs[(it + 1) % 3]

        def _mv_body(i):
            s = ei_v[0, pl.ds(i * L, L)]
            d = ei_v[1, pl.ds(i * L, L)]
            srow = jnp.right_shift(s, 7)
            scol = jnp.bitwise_and(s, 127)
            drow = jnp.right_shift(d, 7)
            dcol = jnp.bitwise_and(d, 127)
            vals = plsc.load_gather(v_v, [drow, dcol]) * we_v[pl.ds(i * L, L)]
            plsc.addupdate_scatter(acc_v, [srow, scol], vals)

        edges_loop(_mv_body)

        cp_add = pltpu.async_copy(acc_v, out_sh.at[idx80_v], dma_sem,
                                  add=True)
        if it < K:
            pltpu.sync_copy(zrows_v, nxt_sh.at[pl.ds(rbase, RPW)])
        cp_add.wait()
        plsc.subcore_barrier()
        if it < K:
            cp_read = pltpu.async_copy(out_sh, v_v, dma_sem2)
            zero_acc()
            cp_read.wait()
        else:
            pltpu.sync_copy(out_sh.at[pl.ds(rbase, RPW)],
                            v_v.at[pl.ds(rbase, RPW)])

        cdamp = (1.0 - ALPHA) ** it
        coef = ALPHA * cdamp if it < K else cdamp

        @plsc.parallel_loop(0, SC_, unroll=5)
        def _(j):
            r = rbase + j // 8
            c = (j % 8) * L
            wsl_v[pl.ds(j * L, L)] = (wsl_v[pl.ds(j * L, L)] +
                                      coef * v_v[r, pl.ds(c, L)])

    pltpu.sync_copy(wsl_v, w_hbm.at[pl.ds(wid * S_PER, S_PER)])


_sc_propagate = functools.partial(
    pl.kernel,
    out_type=jax.ShapeDtypeStruct((N_PAD,), jnp.float32),
    mesh=plsc.VectorSubcoreMesh(
        core_axis_name="c", subcore_axis_name="s", num_cores=1),
    compiler_params=pltpu.CompilerParams(needs_layout_passes=False),
    scratch_types=[
        pltpu.VMEM((2, EW), jnp.int32),
        pltpu.VMEM((EW,), jnp.float32),
        pltpu.VMEM((NR, 128), jnp.float32),
        pltpu.VMEM((NR, 128), jnp.float32),
        pltpu.VMEM((NR, 128), jnp.float32),
        pltpu.VMEM((NR,), jnp.int32),
        pltpu.VMEM((RPW, 128), jnp.float32),
        pltpu.VMEM((S_PER,), jnp.float32),
        pltpu.VMEM_SHARED((NR, 128), jnp.float32),
        pltpu.VMEM_SHARED((NR, 128), jnp.float32),
        pltpu.VMEM_SHARED((NR, 128), jnp.float32),
        pltpu.SemaphoreType.DMA,
        pltpu.SemaphoreType.DMA,
    ],
)(_sc_propagate_body)


R = 2000
G = N // R


def _tc_mlp_body(x_ref, w0_ref, b0_ref, w1_ref, b1_ref,
                 w2_ref, b2_ref, h_ref):
    h = jnp.maximum(x_ref[...] @ w0_ref[...] + b0_ref[...], 0.0)
    h = jnp.maximum(h @ w1_ref[...] + b1_ref[...], 0.0)
    h = jnp.maximum(h @ w2_ref[...] + b2_ref[...], 0.0)
    h_ref[...] = h.astype(jnp.bfloat16)


def _tc_mlp(x, W0, b0, W1, b1, W2, b2):
    return pl.pallas_call(
        _tc_mlp_body,
        grid=(G,),
        in_specs=[
            pl.BlockSpec((R, D), lambda i: (i, 0)),
            pl.BlockSpec((D, H), lambda i: (0, 0)),
            pl.BlockSpec((1, H), lambda i: (0, 0)),
            pl.BlockSpec((H, H), lambda i: (0, 0)),
            pl.BlockSpec((1, H), lambda i: (0, 0)),
            pl.BlockSpec((H, H), lambda i: (0, 0)),
            pl.BlockSpec((1, H), lambda i: (0, 0)),
        ],
        out_specs=pl.BlockSpec((R, H), lambda i: (i, 0)),
        out_shape=jax.ShapeDtypeStruct((N, H), jnp.bfloat16),
        compiler_params=pltpu.CompilerParams(
            dimension_semantics=("arbitrary",)),
    )(x, W0, b0, W1, b1, W2, b2)


def _tc_readout_body(h_ref, w_ref, wc_ref, bc_ref, out_ref, acc_ref):
    i = pl.program_id(0)
    part = jax.lax.dot(w_ref[0].astype(jnp.bfloat16), h_ref[...],
                       preferred_element_type=jnp.float32)

    @pl.when(i == 0)
    def _():
        acc_ref[...] = part

    @pl.when(i > 0)
    def _():
        acc_ref[...] = acc_ref[...] + part

    @pl.when(i == G - 1)
    def _():
        out_ref[...] = acc_ref[...] @ wc_ref[...] + bc_ref[...]


def _tc_readout(h, w3, Wc, bc):
    return pl.pallas_call(
        _tc_readout_body,
        grid=(G,),
        in_specs=[
            pl.BlockSpec((R, H), lambda i: (i, 0)),
            pl.BlockSpec((1, 1, R), lambda i: (i, 0, 0)),
            pl.BlockSpec((H, C), lambda i: (0, 0)),
            pl.BlockSpec((1, C), lambda i: (0, 0)),
        ],
        out_specs=pl.BlockSpec((1, C), lambda i: (0, 0)),
        out_shape=jax.ShapeDtypeStruct((1, C), jnp.float32),
        scratch_shapes=[pltpu.VMEM((1, H), jnp.float32)],
        compiler_params=pltpu.CompilerParams(
            dimension_semantics=("arbitrary",)),
    )(h, w3, Wc, bc)


def kernel(x, edge_index, W0, b0, W1, b1, W2, b2, Wc, bc):
    w_full = _sc_propagate(edge_index)
    h = _tc_mlp(x, W0, b0.reshape(1, H), W1, b1.reshape(1, H),
                W2, b2.reshape(1, H))
    w3 = w_full[:N].reshape(G, 1, R)
    return _tc_readout(h, w3, Wc, bc.reshape(1, C))


# speedup vs baseline: 1.0224x; 1.0224x over previous
import functools

import jax
import jax.numpy as jnp
from jax import lax
from jax.experimental import pallas as pl
from jax.experimental.pallas import tpu as pltpu
from jax.experimental.pallas import tpu_sc as plsc

N = 10000
E = 160000
D = 256
H = 256
C = 10
K = 10
ALPHA = 0.1

NW = 16
E_PER = E // NW
N_PAD = 10240
S_PER = N_PAD // NW
L = 16


def _rsqrt16(x):
    i = plsc.bitcast(x, jnp.int32)
    i = jnp.int32(0x5F3759DF) - jnp.right_shift(i, 1)
    y = plsc.bitcast(i, jnp.float32)
    for _ in range(3):
        y = y * (1.5 - 0.5 * x * y * y)
    return y


NR = 80
RPW = NR // NW


def _sc_propagate_body(ei_hbm, w_hbm,
                       src_v, dst_v, we_v, v_v, acc_v, nrm_v,
                       idx80_v, zrows_v, wsl_v, buf0_sh, buf1_sh, buf2_sh,
                       dma_sem, dma_sem2):
    wid = lax.axis_index("s")
    ebase = wid * E_PER
    rbase = wid * RPW
    EC = E_PER // L
    SC_ = S_PER // L
    bufs = [buf0_sh, buf1_sh, buf2_sh]

    cp_dst = pltpu.async_copy(ei_hbm.at[pl.ds(E + ebase, E_PER)], dst_v,
                              dma_sem)
    cp_src = pltpu.async_copy(ei_hbm.at[pl.ds(ebase, E_PER)], src_v, dma_sem2)

    zeros16 = jnp.zeros((L,), jnp.float32)
    ones16 = jnp.ones((L,), jnp.float32)

    def zero_acc():
        @plsc.parallel_loop(0, NR * 8, unroll=8)
        def _(i):
            acc_v[i // 8, pl.ds((i % 8) * L, L)] = zeros16

    for j in range(RPW):
        idx80_v[pl.ds(j * L, L)] = lax.iota(jnp.int32, L) + j * L

    @plsc.parallel_loop(0, RPW * 8, unroll=8)
    def _(i):
        zrows_v[i // 8, pl.ds((i % 8) * L, L)] = zeros16

    zero_acc()
    pltpu.sync_copy(zrows_v, buf0_sh.at[pl.ds(rbase, RPW)])
    pltpu.sync_copy(zrows_v, buf1_sh.at[pl.ds(rbase, RPW)])

    cp_dst.wait()

    @plsc.parallel_loop(0, EC, unroll=5)
    def _(i):
        d = dst_v[pl.ds(i * L, L)]
        drow = jnp.right_shift(d, 7)
        dcol = jnp.bitwise_and(d, 127)
        plsc.addupdate_scatter(acc_v, [drow, dcol], ones16)

    plsc.subcore_barrier()
    pltpu.sync_copy(acc_v, buf0_sh.at[idx80_v], add=True)
    plsc.subcore_barrier()
    cp_deg = pltpu.async_copy(buf0_sh, v_v, dma_sem)
    zero_acc()
    cp_deg.wait()

    @plsc.parallel_loop(0, NR * 8, unroll=4)
    def _(i):
        r = i // 8
        c = (i % 8) * L
        nrm_v[r, pl.ds(c, L)] = _rsqrt16(jnp.maximum(v_v[r, pl.ds(c, L)], 1.0))

    cp_src.wait()

    @plsc.parallel_loop(0, EC, unroll=4)
    def _(i):
        s = src_v[pl.ds(i * L, L)]
        d = dst_v[pl.ds(i * L, L)]
        srow = jnp.right_shift(s, 7)
        scol = jnp.bitwise_and(s, 127)
        drow = jnp.right_shift(d, 7)
        dcol = jnp.bitwise_and(d, 127)
        we = (plsc.load_gather(nrm_v, [srow, scol]) *
              plsc.load_gather(nrm_v, [drow, dcol]))
        we_v[pl.ds(i * L, L)] = we
        plsc.addupdate_scatter(acc_v, [srow, scol], we * (1.0 / N))

    cp_add1 = pltpu.async_copy(acc_v, buf1_sh.at[idx80_v], dma_sem, add=True)
    pltpu.sync_copy(zrows_v, buf2_sh.at[pl.ds(rbase, RPW)])
    cp_add1.wait()
    plsc.subcore_barrier()
    cp_rd1 = pltpu.async_copy(buf1_sh, v_v, dma_sem2)
    zero_acc()
    cp_rd1.wait()

    cdamp = 1.0 - ALPHA
    coef = ALPHA * cdamp if K > 1 else cdamp

    @plsc.parallel_loop(0, SC_, unroll=5)
    def _(j):
        r = rbase + j // 8
        c = (j % 8) * L
        wsl_v[pl.ds(j * L, L)] = ALPHA * (1.0 / N) + coef * v_v[r, pl.ds(c, L)]

    for it in range(2, K + 1):
        out_sh = bufs[it % 3]
        nxt_sh = bufs[(it + 1) % 3]

        @plsc.parallel_loop(0, EC, unroll=4)
        def _(i):
            s = src_v[pl.ds(i * L, L)]
            d = dst_v[pl.ds(i * L, L)]
            srow = jnp.right_shift(s, 7)
            scol = jnp.bitwise_and(s, 127)
            drow = jnp.right_shift(d, 7)
            dcol = jnp.bitwise_and(d, 127)
            vals = plsc.load_gather(v_v, [drow, dcol]) * we_v[pl.ds(i * L, L)]
            plsc.addupdate_scatter(acc_v, [srow, scol], vals)

        cp_add = pltpu.async_copy(acc_v, out_sh.at[idx80_v], dma_sem,
                                  add=True)
        if it < K:
            pltpu.sync_copy(zrows_v, nxt_sh.at[pl.ds(rbase, RPW)])
        cp_add.wait()
        plsc.subcore_barrier()
        if it < K:
            cp_read = pltpu.async_copy(out_sh, v_v, dma_sem2)
            zero_acc()
            cp_read.wait()
        else:
            pltpu.sync_copy(out_sh.at[pl.ds(rbase, RPW)],
                            v_v.at[pl.ds(rbase, RPW)])

        cdamp = (1.0 - ALPHA) ** it
        coef = ALPHA * cdamp if it < K else cdamp

        @plsc.parallel_loop(0, SC_, unroll=5)
        def _(j):
            r = rbase + j // 8
            c = (j % 8) * L
            wsl_v[pl.ds(j * L, L)] = (wsl_v[pl.ds(j * L, L)] +
                                      coef * v_v[r, pl.ds(c, L)])

    pltpu.sync_copy(wsl_v, w_hbm.at[pl.ds(wid * S_PER, S_PER)])


_sc_propagate = functools.partial(
    pl.kernel,
    out_type=jax.ShapeDtypeStruct((N_PAD,), jnp.float32),
    mesh=plsc.VectorSubcoreMesh(
        core_axis_name="c", subcore_axis_name="s", num_cores=1),
    compiler_params=pltpu.CompilerParams(needs_layout_passes=False),
    scratch_types=[
        pltpu.VMEM((E_PER,), jnp.int32),
        pltpu.VMEM((E_PER,), jnp.int32),
        pltpu.VMEM((E_PER,), jnp.float32),
        pltpu.VMEM((NR, 128), jnp.float32),
        pltpu.VMEM((NR, 128), jnp.float32),
        pltpu.VMEM((NR, 128), jnp.float32),
        pltpu.VMEM((NR,), jnp.int32),
        pltpu.VMEM((RPW, 128), jnp.float32),
        pltpu.VMEM((S_PER,), jnp.float32),
        pltpu.VMEM_SHARED((NR, 128), jnp.float32),
        pltpu.VMEM_SHARED((NR, 128), jnp.float32),
        pltpu.VMEM_SHARED((NR, 128), jnp.float32),
        pltpu.SemaphoreType.DMA,
        pltpu.SemaphoreType.DMA,
    ],
)(_sc_propagate_body)


R = 2000
G = N // R


def _tc_mlp_body(x_ref, w0_ref, b0_ref, w1_ref, b1_ref,
                 w2_ref, b2_ref, h_ref):
    h = jnp.maximum(x_ref[...] @ w0_ref[...] + b0_ref[...], 0.0)
    h = jnp.maximum(h @ w1_ref[...] + b1_ref[...], 0.0)
    h = jnp.maximum(h @ w2_ref[...] + b2_ref[...], 0.0)
    h_ref[...] = h.astype(jnp.bfloat16)


def _tc_mlp(x, W0, b0, W1, b1, W2, b2):
    return pl.pallas_call(
        _tc_mlp_body,
        grid=(G,),
        in_specs=[
            pl.BlockSpec((R, D), lambda i: (i, 0)),
            pl.BlockSpec((D, H), lambda i: (0, 0)),
            pl.BlockSpec((1, H), lambda i: (0, 0)),
            pl.BlockSpec((H, H), lambda i: (0, 0)),
            pl.BlockSpec((1, H), lambda i: (0, 0)),
            pl.BlockSpec((H, H), lambda i: (0, 0)),
            pl.BlockSpec((1, H), lambda i: (0, 0)),
        ],
        out_specs=pl.BlockSpec((R, H), lambda i: (i, 0)),
        out_shape=jax.ShapeDtypeStruct((N, H), jnp.bfloat16),
        compiler_params=pltpu.CompilerParams(
            dimension_semantics=("arbitrary",)),
    )(x, W0, b0, W1, b1, W2, b2)


def _tc_readout_body(h_ref, w_ref, wc_ref, bc_ref, out_ref, acc_ref):
    i = pl.program_id(0)
    part = jax.lax.dot(w_ref[0].astype(jnp.bfloat16), h_ref[...],
                       preferred_element_type=jnp.float32)

    @pl.when(i == 0)
    def _():
        acc_ref[...] = part

    @pl.when(i > 0)
    def _():
        acc_ref[...] = acc_ref[...] + part

    @pl.when(i == G - 1)
    def _():
        out_ref[...] = acc_ref[...] @ wc_ref[...] + bc_ref[...]


def _tc_readout(h, w3, Wc, bc):
    return pl.pallas_call(
        _tc_readout_body,
        grid=(G,),
        in_specs=[
            pl.BlockSpec((R, H), lambda i: (i, 0)),
            pl.BlockSpec((1, 1, R), lambda i: (i, 0, 0)),
            pl.BlockSpec((H, C), lambda i: (0, 0)),
            pl.BlockSpec((1, C), lambda i: (0, 0)),
        ],
        out_specs=pl.BlockSpec((1, C), lambda i: (0, 0)),
        out_shape=jax.ShapeDtypeStruct((1, C), jnp.float32),
        scratch_shapes=[pltpu.VMEM((1, H), jnp.float32)],
        compiler_params=pltpu.CompilerParams(
            dimension_semantics=("arbitrary",)),
    )(h, w3, Wc, bc)


def kernel(x, edge_index, W0, b0, W1, b1, W2, b2, Wc, bc):
    w_full = _sc_propagate(edge_index.reshape(2 * E))
    h = _tc_mlp(x, W0, b0.reshape(1, H), W1, b1.reshape(1, H),
                W2, b2.reshape(1, H))
    w3 = w_full[:N].reshape(G, 1, R)
    return _tc_readout(h, w3, Wc, bc.reshape(1, C))
